# baseline (speedup 1.0000x reference)
import functools

import jax
import jax.numpy as jnp
from jax import lax
from jax.experimental import pallas as pl
from jax.experimental.pallas import tpu as pltpu
from jax.experimental.pallas import tpu_sc as plsc

B = 16
VOCAB = 1000
E = 32
DIM = 32
SPATIAL = DIM * DIM * DIM
EB = 8
NB = E // EB

NC, NS, L = 2, 16, 16
NW = NC * NS
CHUNK = SPATIAL // NW
G = CHUNK // L


def _sc_embed(ids, table_flat):
    mesh = plsc.VectorSubcoreMesh(core_axis_name="c", subcore_axis_name="s")

    @functools.partial(
        pl.kernel,
        mesh=mesh,
        out_type=jax.ShapeDtypeStruct((B, E, DIM, DIM, DIM), jnp.float32),
        compiler_params=pltpu.CompilerParams(needs_layout_passes=False),
        scratch_types=[
            pltpu.VMEM((VOCAB * E,), jnp.float32),
            pltpu.VMEM((2, DIM, DIM), jnp.int32),
            pltpu.VMEM((2, EB, DIM, DIM), jnp.float32),
            pltpu.SemaphoreType.DMA,
            pltpu.SemaphoreType.DMA,
            pltpu.SemaphoreType.DMA,
            pltpu.SemaphoreType.DMA,
        ],
    )
    def k(ids_hbm, tbl_hbm, out_hbm, tbl_v, ids_v, out_v,
          sem_i0, sem_i1, sem_o0, sem_o1):
        sem_i = (sem_i0, sem_i1)
        sem_o = (sem_o0, sem_o1)
        wid = lax.axis_index("s") * NC + lax.axis_index("c")
        pltpu.sync_copy(tbl_hbm, tbl_v)

        def start_ids(b, u):
            pltpu.async_copy(ids_hbm.at[b, wid], ids_v.at[u], sem_i[u])

        def start_out(b, kb, v):
            pltpu.async_copy(
                out_v.at[v], out_hbm.at[b, pl.ds(kb * EB, EB), wid],
                sem_o[v])

        def wait_ids(u):
            pltpu.make_async_copy(
                ids_hbm.at[0, 0], ids_v.at[u], sem_i[u]).wait()

        def wait_out(v):
            pltpu.make_async_copy(
                out_v.at[v], out_hbm.at[0, pl.ds(0, EB), 0], sem_o[v]).wait()

        start_ids(0, 0)
        start_ids(1, 1)

        @pl.loop(0, B, step=2)
        def _(bb):
            for u in range(2):
                b = bb + u
                wait_ids(u)
                for kb in range(NB):
                    v = kb % 2

                    @pl.when(jnp.logical_or(bb > 0, (u * NB + kb) >= 2))
                    def _():
                        wait_out(v)

                    @plsc.parallel_loop(0, G, unroll=2)
                    def _(g):
                        y = g // 2
                        z0 = (g % 2) * L
                        idx = ids_v[u, y, pl.ds(z0, L)]
                        idx = jnp.minimum(jnp.maximum(idx, 0), VOCAB - 1)
                        for el in range(EB):
                            out_v[v, el, y, pl.ds(z0, L)] = plsc.load_gather(
                                tbl_v, [idx + (kb * EB + el) * VOCAB]
                            )

                    start_out(b, kb, v)

                @pl.when(bb < B - 2)
                def _():
                    start_ids(b + 2, u)

        wait_out(0)
        wait_out(1)

    return k(ids, table_flat)


def kernel(parent_blocks, table):
    ids = parent_blocks.astype(jnp.int32)
    return _sc_embed(ids, table.T.reshape(-1))
